# Initial kernel scaffold; baseline (speedup 1.0000x reference)
import jax
import jax.numpy as jnp
from jax.experimental import pallas as pl


def kernel(continuous_f, screen_detections, minimap_detections, items, char_emb, item_emb, Ws1, bs1, Ws2, bs2, Wm1, bm1, Wm2, bm2):
    raise NotImplementedError("write your pallas kernel here")


# speedup vs baseline: 1.2110x; 1.2110x over previous
import functools

import jax
import jax.numpy as jnp
from jax import lax
from jax.experimental import pallas as pl
from jax.experimental.pallas import tpu as pltpu
from jax.experimental.pallas import tpu_sc as plsc

T_ROWS = 50015
EMB = 128
N_DET = 20000
DET_CHUNK = 4000
N_PART = 16
ROWS_PER = 3136
N_ITEMS_GAME = 28672
ITEM_D = 64
ITEM_PER_W = N_ITEMS_GAME // 32


def _scatter_one_table(det_hbm, det_v, claim_v, tab_v, width, ncols, lo):
    lanes = lax.iota(jnp.int32, 16)

    def chunk_body(ci, _):
        c0 = ci * DET_CHUNK
        pltpu.sync_copy(det_hbm.at[pl.ds(c0 * width, DET_CHUNK * width)], det_v)

        def vreg_body(v, _):
            row = v * 16 + lanes
            j = c0 + row
            flat = row * width
            sid = plsc.load_gather(det_v, [flat])
            m = (sid >= lo) & (sid < lo + ROWS_PER)
            rel = jnp.where(m, sid - lo, 0)

            def cond(cc):
                return jnp.any(m & (j > cc))

            def wbody(cc):
                plsc.store_scatter(claim_v, [rel], j, mask=m & (j > cc))
                return plsc.load_gather(claim_v, [rel])

            cc = lax.while_loop(cond, wbody, jnp.full((16,), -1, jnp.int32))
            win = m & (cc == j)
            for k in range(ncols):
                dk = plsc.load_gather(det_v, [flat + (k + 1)])
                plsc.store_scatter(tab_v, [rel + k * ROWS_PER],
                                   dk.astype(jnp.float32), mask=win)
            return 0

        lax.fori_loop(0, DET_CHUNK // 16, vreg_body, 0)
        return 0

    lax.fori_loop(0, N_DET // DET_CHUNK, chunk_body, 0)


def _sc_irregular(screen_det, minimap_det, items, item_emb):
    mesh = plsc.VectorSubcoreMesh(core_axis_name="c", subcore_axis_name="s")

    @functools.partial(
        pl.kernel,
        out_type=(
            jax.ShapeDtypeStruct((N_PART, 4, ROWS_PER), jnp.float32),
            jax.ShapeDtypeStruct((N_PART, 2, ROWS_PER), jnp.float32),
            jax.ShapeDtypeStruct((N_ITEMS_GAME, ITEM_D), jnp.float32),
        ),
        mesh=mesh,
        scratch_types=[
            pltpu.VMEM((DET_CHUNK * 5,), jnp.int32),
            pltpu.VMEM((DET_CHUNK * 3,), jnp.int32),
            pltpu.VMEM((ROWS_PER,), jnp.int32),
            pltpu.VMEM((4 * ROWS_PER,), jnp.float32),
            pltpu.VMEM((ITEM_PER_W,), jnp.int32),
            pltpu.VMEM((ITEM_PER_W, ITEM_D), jnp.float32),
            pltpu.SemaphoreType.DMA,
        ],
        compiler_params=pltpu.CompilerParams(needs_layout_passes=False,
                                             use_tc_tiling_on_sc=False),
    )
    def sc_kernel(sdet_hbm, mdet_hbm, items_hbm, emb_hbm,
                  sfT_hbm, mfT_hbm, irows_hbm,
                  dets_v, detm_v, claim_v, tab_v, idx_v, rows_v, sem):
        c = lax.axis_index("c")
        s = lax.axis_index("s")
        wid = s * 2 + c

        ibase = wid * ITEM_PER_W
        pltpu.sync_copy(items_hbm.at[pl.ds(ibase, ITEM_PER_W)], idx_v)
        gcopy = pltpu.async_copy(emb_hbm.at[idx_v], rows_v, sem)

        part = wid & 15
        lo = part * ROWS_PER

        def claim_init(i, _):
            claim_v[pl.ds(i * 16, 16)] = jnp.full((16,), -1, jnp.int32)
            return 0
        lax.fori_loop(0, ROWS_PER // 16, claim_init, 0)

        def tab_init(i, _):
            tab_v[pl.ds(i * 16, 16)] = jnp.zeros((16,), jnp.float32)
            return 0
        lax.fori_loop(0, 4 * ROWS_PER // 16, tab_init, 0)

        @pl.when(wid < 16)
        def _():
            _scatter_one_table(sdet_hbm, dets_v, claim_v, tab_v, 5, 4, lo)
            for k in range(4):
                pltpu.sync_copy(tab_v.at[pl.ds(k * ROWS_PER, ROWS_PER)],
                                sfT_hbm.at[part, k])

        @pl.when(wid >= 16)
        def _():
            _scatter_one_table(mdet_hbm, detm_v, claim_v, tab_v, 3, 2, lo)
            for k in range(2):
                pltpu.sync_copy(tab_v.at[pl.ds(k * ROWS_PER, ROWS_PER)],
                                mfT_hbm.at[part, k])

        gcopy.wait()
        pltpu.sync_copy(rows_v, irows_hbm.at[pl.ds(ibase, ITEM_PER_W)])

    return sc_kernel(screen_det.reshape(-1), minimap_det.reshape(-1),
                     items, item_emb)


def _tc_mlps(char_emb, sfT, mfT, A1s, B1s, bs1, W2s, bs2, A1m, B1m, bm1, W2m, bm2):
    def body(char_ref, sfT_ref, mfT_ref,
             a1s_ref, b1s_ref, bs1_ref, w2s_ref, bs2_ref,
             a1m_ref, b1m_ref, bm1_ref, w2m_ref, bm2_ref,
             so_ref, mo_ref):
        cb = char_ref[...]
        sft = sfT_ref[0]
        mft = mfT_ref[0]
        dn = (((0,), (0,)), ((), ()))
        cs = lax.dot_general(sft, b1s_ref[...], dn,
                             preferred_element_type=jnp.float32)
        h = jnp.dot(cb, a1s_ref[...], preferred_element_type=jnp.float32)
        h = jnp.maximum(h + cs + bs1_ref[...], 0.0)
        so_ref[...] = (jnp.dot(h, w2s_ref[...], preferred_element_type=jnp.float32)
                       + bs2_ref[...])
        cm = lax.dot_general(mft, b1m_ref[...], dn,
                             preferred_element_type=jnp.float32)
        hm = jnp.dot(cb, a1m_ref[...], preferred_element_type=jnp.float32)
        hm = jnp.maximum(hm + cm + bm1_ref[...], 0.0)
        mo_ref[...] = (jnp.dot(hm, w2m_ref[...], preferred_element_type=jnp.float32)
                       + bm2_ref[...])

    full = lambda shape: pl.BlockSpec(shape, lambda i: (0,) * len(shape))
    return pl.pallas_call(
        body,
        grid=(N_PART,),
        in_specs=[
            pl.BlockSpec((ROWS_PER, EMB), lambda i: (i, 0)),
            pl.BlockSpec((1, 4, ROWS_PER), lambda i: (i, 0, 0)),
            pl.BlockSpec((1, 2, ROWS_PER), lambda i: (i, 0, 0)),
            full((EMB, EMB)), full((4, EMB)), full((1, EMB)), full((EMB, EMB)), full((1, EMB)),
            full((EMB, EMB)), full((2, EMB)), full((1, EMB)), full((EMB, EMB)), full((1, EMB)),
        ],
        out_specs=[
            pl.BlockSpec((ROWS_PER, EMB), lambda i: (i, 0)),
            pl.BlockSpec((ROWS_PER, EMB), lambda i: (i, 0)),
        ],
        out_shape=[
            jax.ShapeDtypeStruct((T_ROWS, EMB), jnp.float32),
            jax.ShapeDtypeStruct((T_ROWS, EMB), jnp.float32),
        ],
    )(char_emb, sfT, mfT, A1s, B1s, bs1, W2s, bs2, A1m, B1m, bm1, W2m, bm2)


def kernel(continuous_f, screen_detections, minimap_detections, items, char_emb,
           item_emb, Ws1, bs1, Ws2, bs2, Wm1, bm1, Wm2, bm2):
    screen_detections = screen_detections.astype(jnp.int32)
    minimap_detections = minimap_detections.astype(jnp.int32)
    items = items.astype(jnp.int32)

    sfT, mfT, irows = _sc_irregular(screen_detections, minimap_detections,
                                    items, item_emb)

    A1s = Ws1[:, :EMB].T
    B1s = Ws1[:, EMB:].T
    W2s = Ws2.T
    A1m = Wm1[:, :EMB].T
    B1m = Wm1[:, EMB:].T
    W2m = Wm2.T

    so, mo = _tc_mlps(char_emb, sfT, mfT,
                      A1s, B1s, bs1.reshape(1, EMB), W2s, bs2.reshape(1, EMB),
                      A1m, B1m, bm1.reshape(1, EMB), W2m, bm2.reshape(1, EMB))

    return jnp.concatenate([
        continuous_f.reshape(1, -1),
        so.reshape(1, -1),
        mo.reshape(1, -1),
        irows.reshape(1, -1),
    ], axis=1)
